# baseline (speedup 1.0000x reference)
import jax
import jax.numpy as jnp
from jax.experimental import pallas as pl
from jax.experimental.pallas import tpu as pltpu

_N = 262144
_B = 64
_M = 64
_TN = 8192
_CH = 512
_LOG2E = 1.4426950408889634
_LN_EPS = -36.841361487904734


def _softplus(x):
    return jnp.logaddexp(x, 0.0)


def _prep_kernel(key_ref, strength_ref, sharpen_ref, q_ref, beta_ref,
                 gamma_ref):
    key = key_ref[...]
    beta = _softplus(strength_ref[...])
    key_n = key / (jnp.sqrt(jnp.sum(key * key, axis=1, keepdims=True)) + 1e-16)
    q_ref[...] = beta * key_n
    beta_ref[...] = beta
    gamma_ref[...] = 1.0 + _softplus(sharpen_ref[...])


def _split_bf16(x):
    hi = x.astype(jnp.bfloat16)
    lo = (x - hi.astype(jnp.float32)).astype(jnp.bfloat16)
    return hi, lo


def _block_scaled_logits(q_ref, mem):
    q_hi, q_lo = _split_bf16(q_ref[...])
    m_hi, m_lo = _split_bf16(mem)
    dn = (((1,), (1,)), ((), ()))
    kw = dict(preferred_element_type=jnp.float32)
    dots = (jax.lax.dot_general(q_hi, m_hi, dn, **kw)
            + jax.lax.dot_general(q_hi, m_lo, dn, **kw)
            + jax.lax.dot_general(q_lo, m_hi, dn, **kw))
    ones = jnp.ones((1, _M), jnp.float32)
    nsq = jax.lax.dot_general(
        ones, mem * mem, (((1,), (1,)), ((), ())),
        preferred_element_type=jnp.float32)
    rl = _LOG2E / (jnp.sqrt(nsq) + 1e-16)
    return dots, rl


def _stats_kernel(q_ref, beta_ref, gamma_ref, mem_ref, s1_ref, s2_ref,
                  acc1_ref, acc2_ref):
    j = pl.program_id(0)
    dots, rl = _block_scaled_logits(q_ref, mem_ref[...])
    beta = beta_ref[...]
    gamma = gamma_ref[...]
    c1 = -beta * _LOG2E
    c2 = gamma * c1
    a1 = jnp.zeros((_B, _CH), jnp.float32)
    a2 = jnp.zeros((_B, _CH), jnp.float32)
    for c in range(_TN // _CH):
        sl = slice(c * _CH, (c + 1) * _CH)
        m1 = dots[:, sl] * rl[:, sl]
        a1 += jnp.exp2(m1 + c1)
        a2 += jnp.exp2(gamma * m1 + c2)
    p1 = jnp.sum(a1, axis=1, keepdims=True)
    p2 = jnp.sum(a2, axis=1, keepdims=True)

    @pl.when(j == 0)
    def _():
        acc1_ref[...] = p1
        acc2_ref[...] = p2

    @pl.when(j > 0)
    def _():
        acc1_ref[...] += p1
        acc2_ref[...] += p2

    @pl.when(j == pl.num_programs(0) - 1)
    def _():
        s1_ref[...] = acc1_ref[...]
        s2_ref[...] = acc2_ref[...]


def _out_kernel(q_ref, beta_ref, gamma_ref, s1_ref, s2_ref, mem_ref, out_ref):
    dots, rl = _block_scaled_logits(q_ref, mem_ref[...])
    beta = beta_ref[...]
    gamma = gamma_ref[...]
    eps_term = jnp.exp(gamma * jnp.log(s1_ref[...]) + _LN_EPS)
    d = s2_ref[...] + eps_term
    c = gamma * (-beta * _LOG2E) - jnp.log2(d)
    for ci in range(_TN // _CH):
        sl = slice(ci * _CH, (ci + 1) * _CH)
        m1 = dots[:, sl] * rl[:, sl]
        out_ref[:, sl] = jnp.exp2(gamma * m1 + c)


@jax.jit
def kernel(key, strength, sharpen, memory):
    col = jax.ShapeDtypeStruct((_B, 1), jnp.float32)
    q, beta, gamma = pl.pallas_call(
        _prep_kernel,
        out_shape=[jax.ShapeDtypeStruct((_B, _M), jnp.float32), col, col],
    )(key, strength, sharpen)

    grid = (_N // _TN,)
    small = [
        pl.BlockSpec((_B, _M), lambda j: (0, 0)),
        pl.BlockSpec((_B, 1), lambda j: (0, 0)),
        pl.BlockSpec((_B, 1), lambda j: (0, 0)),
    ]
    mem_spec = pl.BlockSpec((_TN, _M), lambda j: (j, 0))
    stat_spec = pl.BlockSpec((_B, 1), lambda j: (0, 0))

    s1, s2 = pl.pallas_call(
        _stats_kernel,
        grid=grid,
        in_specs=small + [mem_spec],
        out_specs=[stat_spec, stat_spec],
        out_shape=[col, col],
        scratch_shapes=[pltpu.VMEM((_B, 1), jnp.float32)] * 2,
    )(q, beta, gamma, memory)

    out = pl.pallas_call(
        _out_kernel,
        grid=grid,
        in_specs=small + [stat_spec, stat_spec, mem_spec],
        out_specs=pl.BlockSpec((_B, _TN), lambda j: (0, j)),
        out_shape=jax.ShapeDtypeStruct((_B, _N), jnp.float32),
    )(q, beta, gamma, s1, s2, memory)
    return out


# speedup vs baseline: 1.0501x; 1.0501x over previous
import jax
import jax.numpy as jnp
from jax.experimental import pallas as pl
from jax.experimental.pallas import tpu as pltpu

_N = 262144
_B = 64
_M = 64
_TN = 8192
_G = _N // _TN
_CH = 512
_LOG2E = 1.4426950408889634
_LN_EPS = -36.841361487904734


def _softplus(x):
    return jnp.logaddexp(x, 0.0)


def _split_bf16(x):
    hi = x.astype(jnp.bfloat16)
    lo = (x - hi.astype(jnp.float32)).astype(jnp.bfloat16)
    return hi, lo


def _fused_kernel(key_ref, strength_ref, sharpen_ref, mem_ref, out_ref,
                  qhi_ref, qlo_ref, c1_ref, c2_ref, gamma_ref, acc1_ref,
                  acc2_ref, cden_ref, rl_ref):
    j = pl.program_id(0)

    @pl.when(j == 0)
    def _prep():
        key = key_ref[...]
        beta = _softplus(strength_ref[...])
        gamma = 1.0 + _softplus(sharpen_ref[...])
        key_n = key / (jnp.sqrt(jnp.sum(key * key, axis=1, keepdims=True))
                       + 1e-16)
        q_hi, q_lo = _split_bf16(beta * key_n)
        qhi_ref[...] = q_hi
        qlo_ref[...] = q_lo
        gamma_ref[...] = gamma
        c1_ref[...] = -beta * _LOG2E
        c2_ref[...] = gamma * (-beta) * _LOG2E

    def _dots(mem):
        m_hi, m_lo = _split_bf16(mem)
        q_hi = qhi_ref[...]
        q_lo = qlo_ref[...]
        dn = (((1,), (1,)), ((), ()))
        kw = dict(preferred_element_type=jnp.float32)
        return (jax.lax.dot_general(q_hi, m_hi, dn, **kw)
                + jax.lax.dot_general(q_hi, m_lo, dn, **kw)
                + jax.lax.dot_general(q_lo, m_hi, dn, **kw))

    @pl.when(j < _G)
    def _stats_phase():
        mem = mem_ref[...]
        dots = _dots(mem)
        ones = jnp.ones((1, _M), jnp.float32)
        nsq = jax.lax.dot_general(
            ones, mem * mem, (((1,), (1,)), ((), ())),
            preferred_element_type=jnp.float32)
        rl = _LOG2E / (jnp.sqrt(nsq) + 1e-16)
        rl_ref[0, pl.ds(j * _TN, _TN)] = rl[0, :]
        c1 = c1_ref[...]
        c2 = c2_ref[...]
        gamma = gamma_ref[...]
        a1 = jnp.zeros((_B, _CH), jnp.float32)
        a2 = jnp.zeros((_B, _CH), jnp.float32)
        for c in range(_TN // _CH):
            sl = slice(c * _CH, (c + 1) * _CH)
            m1 = dots[:, sl] * rl[:, sl]
            a1 += jnp.exp2(m1 + c1)
            a2 += jnp.exp2(gamma * m1 + c2)
        p1 = jnp.sum(a1, axis=1, keepdims=True)
        p2 = jnp.sum(a2, axis=1, keepdims=True)

        @pl.when(j == 0)
        def _():
            acc1_ref[...] = p1
            acc2_ref[...] = p2

        @pl.when(j > 0)
        def _():
            acc1_ref[...] += p1
            acc2_ref[...] += p2

    @pl.when(j == _G - 1)
    def _finalize():
        s1 = acc1_ref[...]
        s2 = acc2_ref[...]
        gamma = gamma_ref[...]
        eps_term = jnp.exp(gamma * jnp.log(s1) + _LN_EPS)
        cden_ref[...] = c2_ref[...] - jnp.log2(s2 + eps_term)

    @pl.when(j >= _G)
    def _out_phase():
        dots = _dots(mem_ref[...])
        gamma = gamma_ref[...]
        cden = cden_ref[...]
        rl = rl_ref[0, pl.ds((j - _G) * _TN, _TN)].reshape(1, _TN)
        for c in range(_TN // _CH):
            sl = slice(c * _CH, (c + 1) * _CH)
            m1 = dots[:, sl] * rl[:, sl]
            out_ref[:, sl] = jnp.exp2(gamma * m1 + cden)


@jax.jit
def kernel(key, strength, sharpen, memory):
    return pl.pallas_call(
        _fused_kernel,
        grid=(2 * _G,),
        in_specs=[
            pl.BlockSpec((_B, _M), lambda j: (0, 0)),
            pl.BlockSpec((_B, 1), lambda j: (0, 0)),
            pl.BlockSpec((_B, 1), lambda j: (0, 0)),
            pl.BlockSpec((_TN, _M), lambda j: (j % _G, 0)),
        ],
        out_specs=pl.BlockSpec((_B, _TN),
                               lambda j: (0, jnp.maximum(j - _G, 0))),
        out_shape=jax.ShapeDtypeStruct((_B, _N), jnp.float32),
        scratch_shapes=[
            pltpu.VMEM((_B, _M), jnp.bfloat16),
            pltpu.VMEM((_B, _M), jnp.bfloat16),
            pltpu.VMEM((_B, 1), jnp.float32),
            pltpu.VMEM((_B, 1), jnp.float32),
            pltpu.VMEM((_B, 1), jnp.float32),
            pltpu.VMEM((_B, 1), jnp.float32),
            pltpu.VMEM((_B, 1), jnp.float32),
            pltpu.VMEM((_B, 1), jnp.float32),
            pltpu.VMEM((1, _N), jnp.float32),
        ],
    )(key, strength, sharpen, memory)
